# baseline (speedup 1.0000x reference)
import functools

import jax
import jax.numpy as jnp
from jax import lax
from jax.experimental import pallas as pl
from jax.experimental.pallas import tpu as pltpu
from jax.experimental.pallas import tpu_sc as plsc

_B, _L, _D = 4, 2048, 1024
_NC, _NS = 2, 16
_NW = _NC * _NS
_LS = 512
_RPW = _LS // _NW
_CROWS = 4
_NCH = _RPW // _CROWS
_UNROLL = 4
_LB = 256


def _sc_add_head(x, enc):
    mesh = plsc.VectorSubcoreMesh(core_axis_name="c", subcore_axis_name="s")

    @functools.partial(
        pl.kernel,
        out_type=jax.ShapeDtypeStruct((_B, _LS, _D), jnp.float32),
        mesh=mesh,
        scratch_types=[
            [pltpu.VMEM((_CROWS, _D), jnp.float32) for _ in range(5)],
            [pltpu.VMEM((_CROWS, _D), jnp.float32) for _ in range(5)],
            [pltpu.SemaphoreType.DMA for _ in range(2)],
            [pltpu.SemaphoreType.DMA for _ in range(2)],
        ],
    )
    def k(x_hbm, enc_hbm, out_hbm, set0, set1, lsem, ssem):
        sets = (set0, set1)
        wid = lax.axis_index("s") * _NC + lax.axis_index("c")
        base = wid * _RPW

        def start_loads(i):
            s = i % 2
            bufs = sets[s]
            row0 = base + i * _CROWS
            ds = [pltpu.async_copy(enc_hbm.at[pl.ds(row0, _CROWS)], bufs[0], lsem[s])]
            for b in range(_B):
                ds.append(
                    pltpu.async_copy(x_hbm.at[b, pl.ds(row0, _CROWS)], bufs[1 + b], lsem[s])
                )
            return ds

        loads = {0: start_loads(0)}
        stores = {}
        for i in range(_NCH):
            s = i % 2
            bufs = sets[s]
            row0 = base + i * _CROWS
            if i + 1 < _NCH:
                if (i - 1) in stores:
                    for d in stores.pop(i - 1):
                        d.wait()
                loads[i + 1] = start_loads(i + 1)
            for d in loads.pop(i):
                d.wait()
            for r in range(_CROWS):
                def cbody(j, _, r=r, bufs=bufs):
                    for u in range(_UNROLL):
                        sl = pl.ds((j * _UNROLL + u) * 16, 16)
                        e = bufs[0][r, sl]
                        for b in range(_B):
                            bufs[1 + b][r, sl] = bufs[1 + b][r, sl] + e
                    return 0

                lax.fori_loop(0, _D // (16 * _UNROLL), cbody, 0)
            stores[i] = [
                pltpu.async_copy(bufs[1 + b], out_hbm.at[b, pl.ds(row0, _CROWS)], ssem[s])
                for b in range(_B)
            ]
        for sds in stores.values():
            for d in sds:
                d.wait()

    return k(x, enc)


def _tc_body(x_ref, enc_ref, out_ref):
    out_ref[...] = x_ref[...] + enc_ref[...][None]


def _tc_add_tail(x, enc):
    off = _LS // _LB
    return pl.pallas_call(
        _tc_body,
        grid=((_L - _LS) // _LB,),
        in_specs=[
            pl.BlockSpec((_B, _LB, _D), lambda i: (0, i + off, 0)),
            pl.BlockSpec((_LB, _D), lambda i: (i + off, 0)),
        ],
        out_specs=pl.BlockSpec((_B, _LB, _D), lambda i: (0, i + off, 0)),
        out_shape=jax.ShapeDtypeStruct((_B, _L, _D), jnp.float32),
    )(x, enc)


def kernel(x, encoding):
    enc = encoding[:_L]
    out_sc = _sc_add_head(x, enc)
    out_full = _tc_add_tail(x, enc)
    return lax.dynamic_update_slice(out_full, out_sc, (0, 0, 0))


# speedup vs baseline: 1.0811x; 1.0811x over previous
import functools

import jax
import jax.numpy as jnp
from jax import lax
from jax.experimental import pallas as pl
from jax.experimental.pallas import tpu as pltpu
from jax.experimental.pallas import tpu_sc as plsc

_B, _L, _D = 4, 2048, 1024
_NC, _NS = 2, 16
_NW = _NC * _NS
_LS = 384
_RPW = _LS // _NW
_CROWS = 4
_NCH = _RPW // _CROWS
_UNROLL = 4
_LB = 256


def _sc_add_head(x, enc):
    mesh = plsc.VectorSubcoreMesh(core_axis_name="c", subcore_axis_name="s")

    @functools.partial(
        pl.kernel,
        out_type=jax.ShapeDtypeStruct((_B, _LS, _D), jnp.float32),
        mesh=mesh,
        scratch_types=[
            [pltpu.VMEM((_CROWS, _D), jnp.float32) for _ in range(5)],
            [pltpu.VMEM((_CROWS, _D), jnp.float32) for _ in range(5)],
            [pltpu.SemaphoreType.DMA for _ in range(2)],
            [pltpu.SemaphoreType.DMA for _ in range(2)],
        ],
    )
    def k(x_hbm, enc_hbm, out_hbm, set0, set1, lsem, ssem):
        sets = (set0, set1)
        wid = lax.axis_index("s") * _NC + lax.axis_index("c")
        base = wid * _RPW

        def start_loads(i):
            s = i % 2
            bufs = sets[s]
            row0 = base + i * _CROWS
            ds = [pltpu.async_copy(enc_hbm.at[pl.ds(row0, _CROWS)], bufs[0], lsem[s])]
            for b in range(_B):
                ds.append(
                    pltpu.async_copy(x_hbm.at[b, pl.ds(row0, _CROWS)], bufs[1 + b], lsem[s])
                )
            return ds

        loads = {0: start_loads(0)}
        stores = {}
        for i in range(_NCH):
            s = i % 2
            bufs = sets[s]
            row0 = base + i * _CROWS
            if i + 1 < _NCH:
                if (i - 1) in stores:
                    for d in stores.pop(i - 1):
                        d.wait()
                loads[i + 1] = start_loads(i + 1)
            for d in loads.pop(i):
                d.wait()
            for r in range(_CROWS):
                def cbody(j, _, r=r, bufs=bufs):
                    for u in range(_UNROLL):
                        sl = pl.ds((j * _UNROLL + u) * 16, 16)
                        e = bufs[0][r, sl]
                        for b in range(_B):
                            bufs[1 + b][r, sl] = bufs[1 + b][r, sl] + e
                    return 0

                lax.fori_loop(0, _D // (16 * _UNROLL), cbody, 0)
            stores[i] = [
                pltpu.async_copy(bufs[1 + b], out_hbm.at[b, pl.ds(row0, _CROWS)], ssem[s])
                for b in range(_B)
            ]
        for sds in stores.values():
            for d in sds:
                d.wait()

    return k(x, enc)


def _tc_body(x_ref, enc_ref, out_ref):
    out_ref[...] = x_ref[...] + enc_ref[...][None]


def _tc_add_tail(x, enc):
    off = _LS // _LB
    return pl.pallas_call(
        _tc_body,
        grid=((_L - _LS) // _LB,),
        in_specs=[
            pl.BlockSpec((_B, _LB, _D), lambda i: (0, i + off, 0)),
            pl.BlockSpec((_LB, _D), lambda i: (i + off, 0)),
        ],
        out_specs=pl.BlockSpec((_B, _LB, _D), lambda i: (0, i + off, 0)),
        out_shape=jax.ShapeDtypeStruct((_B, _L, _D), jnp.float32),
    )(x, enc)


def kernel(x, encoding):
    enc = encoding[:_L]
    out_sc = _sc_add_head(x, enc)
    out_full = _tc_add_tail(x, enc)
    return lax.dynamic_update_slice(out_full, out_sc, (0, 0, 0))


# speedup vs baseline: 1.0813x; 1.0002x over previous
import functools

import jax
import jax.numpy as jnp
from jax import lax
from jax.experimental import pallas as pl
from jax.experimental.pallas import tpu as pltpu
from jax.experimental.pallas import tpu_sc as plsc

_B, _L, _D = 4, 2048, 1024
_NC, _NS = 2, 16
_NW = _NC * _NS
_LS = 256
_RPW = _LS // _NW
_CROWS = 4
_NCH = _RPW // _CROWS
_UNROLL = 4
_LB = 256


def _sc_add_head(x, enc):
    mesh = plsc.VectorSubcoreMesh(core_axis_name="c", subcore_axis_name="s")

    @functools.partial(
        pl.kernel,
        out_type=jax.ShapeDtypeStruct((_B, _LS, _D), jnp.float32),
        mesh=mesh,
        scratch_types=[
            [pltpu.VMEM((_CROWS, _D), jnp.float32) for _ in range(5)],
            [pltpu.VMEM((_CROWS, _D), jnp.float32) for _ in range(5)],
            [pltpu.SemaphoreType.DMA for _ in range(2)],
            [pltpu.SemaphoreType.DMA for _ in range(2)],
        ],
    )
    def k(x_hbm, enc_hbm, out_hbm, set0, set1, lsem, ssem):
        sets = (set0, set1)
        wid = lax.axis_index("s") * _NC + lax.axis_index("c")
        base = wid * _RPW

        def start_loads(i):
            s = i % 2
            bufs = sets[s]
            row0 = base + i * _CROWS
            ds = [pltpu.async_copy(enc_hbm.at[pl.ds(row0, _CROWS)], bufs[0], lsem[s])]
            for b in range(_B):
                ds.append(
                    pltpu.async_copy(x_hbm.at[b, pl.ds(row0, _CROWS)], bufs[1 + b], lsem[s])
                )
            return ds

        loads = {0: start_loads(0)}
        stores = {}
        for i in range(_NCH):
            s = i % 2
            bufs = sets[s]
            row0 = base + i * _CROWS
            if i + 1 < _NCH:
                if (i - 1) in stores:
                    for d in stores.pop(i - 1):
                        d.wait()
                loads[i + 1] = start_loads(i + 1)
            for d in loads.pop(i):
                d.wait()
            for r in range(_CROWS):
                def cbody(j, _, r=r, bufs=bufs):
                    for u in range(_UNROLL):
                        sl = pl.ds((j * _UNROLL + u) * 16, 16)
                        e = bufs[0][r, sl]
                        for b in range(_B):
                            bufs[1 + b][r, sl] = bufs[1 + b][r, sl] + e
                    return 0

                lax.fori_loop(0, _D // (16 * _UNROLL), cbody, 0)
            stores[i] = [
                pltpu.async_copy(bufs[1 + b], out_hbm.at[b, pl.ds(row0, _CROWS)], ssem[s])
                for b in range(_B)
            ]
        for sds in stores.values():
            for d in sds:
                d.wait()

    return k(x, enc)


def _tc_body(x_ref, enc_ref, out_ref):
    out_ref[...] = x_ref[...] + enc_ref[...][None]


def _tc_add_tail(x, enc):
    off = _LS // _LB
    return pl.pallas_call(
        _tc_body,
        grid=((_L - _LS) // _LB,),
        in_specs=[
            pl.BlockSpec((_B, _LB, _D), lambda i: (0, i + off, 0)),
            pl.BlockSpec((_LB, _D), lambda i: (i + off, 0)),
        ],
        out_specs=pl.BlockSpec((_B, _LB, _D), lambda i: (0, i + off, 0)),
        out_shape=jax.ShapeDtypeStruct((_B, _L, _D), jnp.float32),
    )(x, enc)


def kernel(x, encoding):
    enc = encoding[:_L]
    out_sc = _sc_add_head(x, enc)
    out_full = _tc_add_tail(x, enc)
    return lax.dynamic_update_slice(out_full, out_sc, (0, 0, 0))


# speedup vs baseline: 1.1607x; 1.0734x over previous
import functools

import jax
import jax.numpy as jnp
from jax import lax
from jax.experimental import pallas as pl
from jax.experimental.pallas import tpu as pltpu
from jax.experimental.pallas import tpu_sc as plsc

_B, _L, _D = 4, 2048, 1024
_NC, _NS = 2, 16
_NW = _NC * _NS
_LS = 256
_RPW = _LS // _NW
_CROWS = 4
_NCH = _RPW // _CROWS
_UNROLL = 4
_LB = 512


def _sc_add_head(x, enc):
    mesh = plsc.VectorSubcoreMesh(core_axis_name="c", subcore_axis_name="s")

    @functools.partial(
        pl.kernel,
        out_type=jax.ShapeDtypeStruct((_B, _LS, _D), jnp.float32),
        mesh=mesh,
        scratch_types=[
            [pltpu.VMEM((_CROWS, _D), jnp.float32) for _ in range(5)],
            [pltpu.VMEM((_CROWS, _D), jnp.float32) for _ in range(5)],
            [pltpu.SemaphoreType.DMA for _ in range(2)],
            [pltpu.SemaphoreType.DMA for _ in range(2)],
        ],
    )
    def k(x_hbm, enc_hbm, out_hbm, set0, set1, lsem, ssem):
        sets = (set0, set1)
        wid = lax.axis_index("s") * _NC + lax.axis_index("c")
        base = wid * _RPW

        def start_loads(i):
            s = i % 2
            bufs = sets[s]
            row0 = base + i * _CROWS
            ds = [pltpu.async_copy(enc_hbm.at[pl.ds(row0, _CROWS)], bufs[0], lsem[s])]
            for b in range(_B):
                ds.append(
                    pltpu.async_copy(x_hbm.at[b, pl.ds(row0, _CROWS)], bufs[1 + b], lsem[s])
                )
            return ds

        loads = {0: start_loads(0)}
        stores = {}
        for i in range(_NCH):
            s = i % 2
            bufs = sets[s]
            row0 = base + i * _CROWS
            if i + 1 < _NCH:
                if (i - 1) in stores:
                    for d in stores.pop(i - 1):
                        d.wait()
                loads[i + 1] = start_loads(i + 1)
            for d in loads.pop(i):
                d.wait()
            for r in range(_CROWS):
                def cbody(j, _, r=r, bufs=bufs):
                    for u in range(_UNROLL):
                        sl = pl.ds((j * _UNROLL + u) * 16, 16)
                        e = bufs[0][r, sl]
                        for b in range(_B):
                            bufs[1 + b][r, sl] = bufs[1 + b][r, sl] + e
                    return 0

                lax.fori_loop(0, _D // (16 * _UNROLL), cbody, 0)
            stores[i] = [
                pltpu.async_copy(bufs[1 + b], out_hbm.at[b, pl.ds(row0, _CROWS)], ssem[s])
                for b in range(_B)
            ]
        for sds in stores.values():
            for d in sds:
                d.wait()

    return k(x, enc)


def _tc_body(x_ref, enc_ref, out_ref):
    out_ref[...] = x_ref[...] + enc_ref[...][None]


def _tc_add_tail(x, enc):
    off = _LS // _LB
    return pl.pallas_call(
        _tc_body,
        grid=((_L - _LS) // _LB,),
        in_specs=[
            pl.BlockSpec((_B, _LB, _D), lambda i: (0, i + off, 0)),
            pl.BlockSpec((_LB, _D), lambda i: (i + off, 0)),
        ],
        out_specs=pl.BlockSpec((_B, _LB, _D), lambda i: (0, i + off, 0)),
        out_shape=jax.ShapeDtypeStruct((_B, _L, _D), jnp.float32),
    )(x, enc)


def kernel(x, encoding):
    enc = encoding[:_L]
    out_full = _tc_add_tail(x, enc)
    out_sc = _sc_add_head(x, enc)
    return lax.dynamic_update_slice(out_full, out_sc, (0, 0, 0))
